# baseline (speedup 1.0000x reference)
import functools

import jax
import jax.numpy as jnp
from jax import lax
from jax.experimental import pallas as pl
from jax.experimental.pallas import tpu as pltpu
from jax.experimental.pallas import tpu_sc as plsc

NUM_CODES = 8192
CODE_DIM = 256


_BM = 256
_BN = 1024
_N_CHUNKS = NUM_CODES // _BN


def _argmin_body(x_ref, cb_ref, idx_ref):
    xb = x_ref[...]
    x_sq = jnp.sum(xb * xb, axis=1, keepdims=True)

    best_val = jnp.full((_BM, 1), jnp.inf, dtype=jnp.float32)
    best_idx = jnp.zeros((_BM, 1), dtype=jnp.int32)
    for c in range(_N_CHUNKS):
        cb = cb_ref[pl.ds(c * _BN, _BN), :]
        c_sq = jnp.sum(cb * cb, axis=1)[None, :]
        xc = lax.dot_general(
            xb.astype(jnp.bfloat16), cb.astype(jnp.bfloat16),
            (((1,), (1,)), ((), ())),
            preferred_element_type=jnp.float32)
        d = (x_sq + c_sq) - 2.0 * xc
        m = jnp.min(d, axis=1, keepdims=True)
        col = jax.lax.broadcasted_iota(jnp.int32, (_BM, _BN), 1) + (c * _BN)
        cand = jnp.min(jnp.where(d == m, col, jnp.int32(2**31 - 1)),
                       axis=1, keepdims=True)
        take = m < best_val
        best_val = jnp.where(take, m, best_val)
        best_idx = jnp.where(take, cand, best_idx)

    idx_ref[...] = best_idx[:, 0]


def _tc_argmin(flat_x, codebook):
    grid = flat_x.shape[0] // _BM
    return pl.pallas_call(
        _argmin_body,
        grid=(grid,),
        in_specs=[
            pl.BlockSpec((_BM, CODE_DIM), lambda i: (i, 0)),
            pl.BlockSpec((NUM_CODES, CODE_DIM), lambda i: (0, 0)),
        ],
        out_specs=pl.BlockSpec((_BM,), lambda i: (i,)),
        out_shape=jax.ShapeDtypeStruct((flat_x.shape[0],), jnp.int32),
    )(flat_x, codebook)


_GC = 128


def _sc_gather(codebook, indices):
    B = indices.shape[0]
    info = plsc.get_sparse_core_info()
    nc, ns = info.num_cores, info.num_subcores
    nw = nc * ns
    b_per_w = B // nw
    n_chunks = b_per_w // _GC
    mesh = plsc.VectorSubcoreMesh(core_axis_name="c", subcore_axis_name="s")

    @functools.partial(
        pl.kernel,
        mesh=mesh,
        out_type=jax.ShapeDtypeStruct((B, CODE_DIM), jnp.float32),
        scratch_types=[
            pltpu.VMEM((_GC,), jnp.int32),
            pltpu.VMEM((_GC, CODE_DIM), jnp.float32),
            pltpu.SemaphoreType.DMA,
        ],
    )
    def k(table_hbm, idx_hbm, out_hbm, idx_v, rows_v, sem):
        wid = lax.axis_index("s") * nc + lax.axis_index("c")
        base = wid * b_per_w
        for c in range(n_chunks):
            off = base + c * _GC
            pltpu.sync_copy(idx_hbm.at[pl.ds(off, _GC)], idx_v)
            pltpu.async_copy(table_hbm.at[idx_v], rows_v, sem).wait()
            pltpu.sync_copy(rows_v, out_hbm.at[pl.ds(off, _GC)])

    return k(codebook, indices)


def kernel(x, codebook):
    input_shape = x.shape
    flat_x = x.reshape(-1, CODE_DIM)
    indices_flat = _tc_argmin(flat_x, codebook)
    quantized_flat = _sc_gather(codebook, indices_flat)
    quantized = quantized_flat.reshape(input_shape)
    quantized = x + lax.stop_gradient(quantized - x)
    indices = indices_flat.reshape(input_shape[:-1])
    return (quantized, indices)
